# baseline (speedup 1.0000x reference)
import math

import jax
import jax.numpy as jnp
from jax.experimental import pallas as pl
from jax.experimental.pallas import tpu as pltpu

HEADS = 8
NEG = -1e30
SQRT_2_OVER_PI = 0.7978845608028654
BF16 = jnp.bfloat16


def _ln_rows(xb, s, b):
    m = jnp.mean(xb, axis=-1, keepdims=True)
    v = jnp.mean((xb - m) ** 2, axis=-1, keepdims=True)
    return (xb - m) * jax.lax.rsqrt(v + 1e-5) * s + b


def _gelu(u):
    return 0.5 * u * (1.0 + jnp.tanh(SQRT_2_OVER_PI * (u + 0.044715 * u * u * u)))


def _dot(a, b):
    return jnp.dot(a.astype(BF16), b, preferred_element_type=jnp.float32)


def _router_body(x_ref, wr_ref, br_ref, wfc_ref, bfc_ref, temp_ref, g_ref,
                 acc_ref, *, n_rows):
    t = pl.program_id(0)

    @pl.when(t == 0)
    def _():
        acc_ref[...] = jnp.zeros_like(acc_ref)

    hb = jnp.dot(x_ref[...], wr_ref[...], preferred_element_type=jnp.float32)
    hb = jnp.maximum(hb + br_ref[...], 0.0)
    acc_ref[...] += jnp.sum(hb, axis=0, keepdims=True)

    @pl.when(t == pl.num_programs(0) - 1)
    def _():
        r = acc_ref[...] * (1.0 / n_rows)
        logits = jnp.dot(r, wfc_ref[...], preferred_element_type=jnp.float32)
        logits = (logits + bfc_ref[...]) / temp_ref[...]
        m = jnp.max(logits, axis=-1, keepdims=True)
        e = jnp.exp(logits - m)
        g_ref[...] = e / jnp.sum(e, axis=-1, keepdims=True)


def _proj_body(x_ref, wp_ref, bp_ref, cls_ref, h_ref, *, nt):
    t = pl.program_id(1)

    @pl.when(t < nt)
    def _():
        hb = jnp.dot(x_ref[...], wp_ref[0], preferred_element_type=jnp.float32)
        h_ref[0] = jnp.maximum(hb + bp_ref[0], 0.0)

    @pl.when(t == nt)
    def _():
        ri = jax.lax.broadcasted_iota(jnp.int32, h_ref.shape[1:], 0)
        h_ref[0] = jnp.where(ri == 0, cls_ref[0], 0.0)


def _lnqkv_body(h_ref, s_ref, b_ref, w_ref, bias_ref, qkv_ref):
    y = _ln_rows(h_ref[0], s_ref[0], b_ref[0])
    qkv_ref[0] = (_dot(y, w_ref[0]) + bias_ref[0]).astype(BF16)


def _mha(qsrc, kv_slice, addmask, dh, scale):
    heads = []
    for hd in range(HEADS):
        q = qsrc[:, hd * dh:(hd + 1) * dh] * scale
        k = kv_slice(0, hd)
        v = kv_slice(1, hd)
        s = jax.lax.dot_general(q, k, (((1,), (1,)), ((), ())),
                                preferred_element_type=jnp.float32)
        s = s + addmask
        m = jnp.max(s, axis=-1, keepdims=True)
        p = jnp.exp((s - m).astype(BF16))
        l = jnp.sum(p, axis=-1, keepdims=True, dtype=jnp.float32)
        o = jax.lax.dot_general(p, v, (((1,), (0,)), ((), ())),
                                preferred_element_type=jnp.float32)
        heads.append(o * (1.0 / l))
    return jnp.concatenate(heads, axis=1)


def _attn_mlp_body(h_ref, qkvt_ref, qkv_hbm, wo_ref, bo_ref, ln2s_ref,
                   ln2b_ref, w1_ref, b1_ref, w2_ref, b2_ref, o_ref,
                   kv_vmem, sem, *, n_valid, dh):
    e = pl.program_id(0)
    t = pl.program_id(1)

    @pl.when(t == 0)
    def _():
        cp = pltpu.make_async_copy(qkv_hbm.at[e], kv_vmem, sem)
        cp.start()
        cp.wait()

    r_full = kv_vmem.shape[0]
    d = h_ref.shape[2]
    col = jax.lax.broadcasted_iota(jnp.int32, (8, r_full), 1)[0:1]
    addmask = jnp.where(col < n_valid, 0.0, NEG)
    scale = jnp.asarray(1.0 / math.sqrt(dh), BF16)

    def kv_slice(which, hd):
        return kv_vmem[:, (1 + which) * d + hd * (d // HEADS):
                       (1 + which) * d + (hd + 1) * (d // HEADS)]

    o = _mha(qkvt_ref[0], kv_slice, addmask, dh, scale)
    hb = h_ref[0] + _dot(o, wo_ref[0]) + bo_ref[0]
    y = _ln_rows(hb, ln2s_ref[0], ln2b_ref[0])
    u = _gelu(_dot(y, w1_ref[0]) + b1_ref[0])
    o_ref[0] = hb + _dot(u, w2_ref[0]) + b2_ref[0]


def _lnkv_body(h_ref, s_ref, b_ref, w_ref, bias_ref, kv_ref):
    y = _ln_rows(h_ref[0], s_ref[0], b_ref[0])
    kv_ref[0] = (_dot(y, w_ref[0]) + bias_ref[0]).astype(BF16)


def _cls_body(h_ref, kv_ref, ln1s_ref, ln1b_ref, wq_ref, bq_ref, wo_ref,
              bo_ref, ln2s_ref, ln2b_ref, w1_ref, b1_ref, w2_ref, b2_ref,
              lnfs_ref, lnfb_ref, wh_ref, bh_ref, g_ref, lat_ref, log_ref,
              *, n_valid, dh):
    e = pl.program_id(0)
    r_full = kv_ref.shape[1]
    d = h_ref.shape[2]
    col = jax.lax.broadcasted_iota(jnp.int32, (8, r_full), 1)[0:1]
    addmask = jnp.where(col < n_valid, 0.0, NEG)
    scale = jnp.asarray(1.0 / math.sqrt(dh), BF16)

    def kv_slice(which, hd):
        return kv_ref[0, :, which * d + hd * dh:which * d + (hd + 1) * dh]

    h8 = h_ref[0]
    y = _ln_rows(h8, ln1s_ref[0], ln1b_ref[0])
    q = (_dot(y, wq_ref[0]) + bq_ref[0]).astype(BF16)
    o = _mha(q, kv_slice, addmask, dh, scale)
    h8 = h8 + _dot(o, wo_ref[0]) + bo_ref[0]
    y = _ln_rows(h8, ln2s_ref[0], ln2b_ref[0])
    u = _gelu(_dot(y, w1_ref[0]) + b1_ref[0])
    h8 = h8 + _dot(u, w2_ref[0]) + b2_ref[0]
    ln = _ln_rows(h8, lnfs_ref[0], lnfb_ref[0])
    lat = ln[0:1, :]
    lg = _dot(lat, wh_ref[0]) + bh_ref[0]
    ge = g_ref[0]
    contrib_lat = lat * ge
    contrib_log = lg * ge

    @pl.when(e == 0)
    def _():
        lat_ref[...] = contrib_lat
        log_ref[...] = contrib_log

    @pl.when(e > 0)
    def _():
        lat_ref[...] += contrib_lat
        log_ref[...] += contrib_log


def _mlp_body(h_ref, s_ref, b_ref, w1_ref, b1_ref, w2_ref, b2_ref, o_ref):
    hb = h_ref[0]
    y = _ln_rows(hb, s_ref[0], b_ref[0])
    u = _gelu(_dot(y, w1_ref[0]) + b1_ref[0])
    o_ref[0] = hb + _dot(u, w2_ref[0]) + b2_ref[0]


def kernel(x, params, temp=1.0):
    b, n, din = x.shape
    experts = params['experts']
    e_num = len(experts)
    d = experts[0]['proj_W'].shape[-1]
    f3 = 3 * d
    dh = d // HEADS
    c_num = experts[0]['head_W'].shape[-1]
    depth = len(experts[0]['layers'])
    f32 = jnp.float32

    x2 = x.reshape(n, din).astype(BF16)

    bt = 256 if n % 256 == 0 else (128 if n % 128 == 0 else n)
    nt = n // bt
    r_full = nt * bt + bt
    rt = r_full // bt
    n_valid = n + 1

    def st(key):
        return jnp.stack([p[key] for p in experts])

    def stl(layer, key):
        return jnp.stack([p['layers'][layer][key] for p in experts])

    wp = st('proj_W').astype(BF16)
    bp = st('proj_b').reshape(e_num, 1, d)
    cls = st('cls').reshape(e_num, 1, d)
    lnf_s = st('lnf_s').reshape(e_num, 1, d)
    lnf_b = st('lnf_b').reshape(e_num, 1, d)
    head_w = st('head_W').astype(BF16)
    head_b = st('head_b').reshape(e_num, 1, c_num)
    temp_arr = jnp.asarray(temp, f32).reshape(1, 1)

    g = pl.pallas_call(
        lambda *a: _router_body(*a, n_rows=float(n)),
        grid=(nt,),
        in_specs=[
            pl.BlockSpec((bt, din), lambda t: (t, 0)),
            pl.BlockSpec((din, d), lambda t: (0, 0)),
            pl.BlockSpec((1, d), lambda t: (0, 0)),
            pl.BlockSpec((d, e_num), lambda t: (0, 0)),
            pl.BlockSpec((1, e_num), lambda t: (0, 0)),
            pl.BlockSpec((1, 1), lambda t: (0, 0)),
        ],
        out_specs=pl.BlockSpec((1, e_num), lambda t: (0, 0)),
        out_shape=jax.ShapeDtypeStruct((1, e_num), f32),
        scratch_shapes=[pltpu.VMEM((1, d), f32)],
    )(x2, params['router_proj_W'].astype(BF16),
      params['router_proj_b'].reshape(1, d),
      params['router_fc_W'], params['router_fc_b'].reshape(1, e_num), temp_arr)

    h = pl.pallas_call(
        lambda *a: _proj_body(*a, nt=nt),
        grid=(e_num, rt),
        in_specs=[
            pl.BlockSpec((bt, din), lambda e, t: (jnp.minimum(t, nt - 1), 0)),
            pl.BlockSpec((1, din, d), lambda e, t: (e, 0, 0)),
            pl.BlockSpec((1, 1, d), lambda e, t: (e, 0, 0)),
            pl.BlockSpec((1, 1, d), lambda e, t: (e, 0, 0)),
        ],
        out_specs=pl.BlockSpec((1, bt, d), lambda e, t: (e, t, 0)),
        out_shape=jax.ShapeDtypeStruct((e_num, r_full, d), f32),
    )(x2, wp, bp, cls)

    for layer in range(depth - 1):
        ln1_s = stl(layer, 'ln1_s').reshape(e_num, 1, d)
        ln1_b = stl(layer, 'ln1_b').reshape(e_num, 1, d)
        wqkv = stl(layer, 'Wqkv').astype(BF16)
        bqkv = stl(layer, 'bqkv').reshape(e_num, 1, f3)
        wo = stl(layer, 'Wo').astype(BF16)
        bo = stl(layer, 'bo').reshape(e_num, 1, d)
        ln2_s = stl(layer, 'ln2_s').reshape(e_num, 1, d)
        ln2_b = stl(layer, 'ln2_b').reshape(e_num, 1, d)
        w1 = stl(layer, 'W1').astype(BF16)
        b1 = stl(layer, 'b1').reshape(e_num, 1, w1.shape[-1])
        w2 = stl(layer, 'W2').astype(BF16)
        b2 = stl(layer, 'b2').reshape(e_num, 1, d)
        mlp_d = w1.shape[-1]

        qkv = pl.pallas_call(
            _lnqkv_body,
            grid=(e_num, rt),
            in_specs=[
                pl.BlockSpec((1, bt, d), lambda e, t: (e, t, 0)),
                pl.BlockSpec((1, 1, d), lambda e, t: (e, 0, 0)),
                pl.BlockSpec((1, 1, d), lambda e, t: (e, 0, 0)),
                pl.BlockSpec((1, d, f3), lambda e, t: (e, 0, 0)),
                pl.BlockSpec((1, 1, f3), lambda e, t: (e, 0, 0)),
            ],
            out_specs=pl.BlockSpec((1, bt, f3), lambda e, t: (e, t, 0)),
            out_shape=jax.ShapeDtypeStruct((e_num, r_full, f3), BF16),
        )(h, ln1_s, ln1_b, wqkv, bqkv)

        h = pl.pallas_call(
            lambda *a: _attn_mlp_body(*a, n_valid=n_valid, dh=dh),
            grid=(e_num, rt),
            in_specs=[
                pl.BlockSpec((1, bt, d), lambda e, t: (e, t, 0)),
                pl.BlockSpec((1, bt, f3), lambda e, t: (e, t, 0)),
                pl.BlockSpec(memory_space=pl.ANY),
                pl.BlockSpec((1, d, d), lambda e, t: (e, 0, 0)),
                pl.BlockSpec((1, 1, d), lambda e, t: (e, 0, 0)),
                pl.BlockSpec((1, 1, d), lambda e, t: (e, 0, 0)),
                pl.BlockSpec((1, 1, d), lambda e, t: (e, 0, 0)),
                pl.BlockSpec((1, d, mlp_d), lambda e, t: (e, 0, 0)),
                pl.BlockSpec((1, 1, mlp_d), lambda e, t: (e, 0, 0)),
                pl.BlockSpec((1, mlp_d, d), lambda e, t: (e, 0, 0)),
                pl.BlockSpec((1, 1, d), lambda e, t: (e, 0, 0)),
            ],
            out_specs=pl.BlockSpec((1, bt, d), lambda e, t: (e, t, 0)),
            out_shape=jax.ShapeDtypeStruct((e_num, r_full, d), f32),
            scratch_shapes=[pltpu.VMEM((r_full, f3), BF16),
                            pltpu.SemaphoreType.DMA],
        )(h, qkv, qkv, wo, bo, ln2_s, ln2_b, w1, b1, w2, b2)

    last = depth - 1
    ln1_s = stl(last, 'ln1_s').reshape(e_num, 1, d)
    ln1_b = stl(last, 'ln1_b').reshape(e_num, 1, d)
    wqkv = stl(last, 'Wqkv').astype(BF16)
    bqkv = stl(last, 'bqkv').reshape(e_num, 1, f3)
    wq_l = wqkv[:, :, :d]
    bq_l = bqkv[:, :, :d]
    wkv_l = wqkv[:, :, d:]
    bkv_l = bqkv[:, :, d:]
    wo = stl(last, 'Wo').astype(BF16)
    bo = stl(last, 'bo').reshape(e_num, 1, d)
    ln2_s = stl(last, 'ln2_s').reshape(e_num, 1, d)
    ln2_b = stl(last, 'ln2_b').reshape(e_num, 1, d)
    w1 = stl(last, 'W1').astype(BF16)
    b1 = stl(last, 'b1').reshape(e_num, 1, w1.shape[-1])
    w2 = stl(last, 'W2').astype(BF16)
    b2 = stl(last, 'b2').reshape(e_num, 1, d)
    mlp_d = w1.shape[-1]
    d2 = 2 * d

    kv = pl.pallas_call(
        _lnkv_body,
        grid=(e_num, rt),
        in_specs=[
            pl.BlockSpec((1, bt, d), lambda e, t: (e, t, 0)),
            pl.BlockSpec((1, 1, d), lambda e, t: (e, 0, 0)),
            pl.BlockSpec((1, 1, d), lambda e, t: (e, 0, 0)),
            pl.BlockSpec((1, d, d2), lambda e, t: (e, 0, 0)),
            pl.BlockSpec((1, 1, d2), lambda e, t: (e, 0, 0)),
        ],
        out_specs=pl.BlockSpec((1, bt, d2), lambda e, t: (e, t, 0)),
        out_shape=jax.ShapeDtypeStruct((e_num, r_full, d2), BF16),
    )(h, ln1_s, ln1_b, wkv_l, bkv_l)

    g3 = g.reshape(e_num, 1, 1)
    cls_row_blk = n // 8
    latent, logits = pl.pallas_call(
        lambda *a: _cls_body(*a, n_valid=n_valid, dh=dh),
        grid=(e_num,),
        in_specs=[
            pl.BlockSpec((1, 8, d), lambda e: (e, cls_row_blk, 0)),
            pl.BlockSpec((1, r_full, d2), lambda e: (e, 0, 0)),
            pl.BlockSpec((1, 1, d), lambda e: (e, 0, 0)),
            pl.BlockSpec((1, 1, d), lambda e: (e, 0, 0)),
            pl.BlockSpec((1, d, d), lambda e: (e, 0, 0)),
            pl.BlockSpec((1, 1, d), lambda e: (e, 0, 0)),
            pl.BlockSpec((1, d, d), lambda e: (e, 0, 0)),
            pl.BlockSpec((1, 1, d), lambda e: (e, 0, 0)),
            pl.BlockSpec((1, 1, d), lambda e: (e, 0, 0)),
            pl.BlockSpec((1, 1, d), lambda e: (e, 0, 0)),
            pl.BlockSpec((1, d, mlp_d), lambda e: (e, 0, 0)),
            pl.BlockSpec((1, 1, mlp_d), lambda e: (e, 0, 0)),
            pl.BlockSpec((1, mlp_d, d), lambda e: (e, 0, 0)),
            pl.BlockSpec((1, 1, d), lambda e: (e, 0, 0)),
            pl.BlockSpec((1, 1, d), lambda e: (e, 0, 0)),
            pl.BlockSpec((1, 1, d), lambda e: (e, 0, 0)),
            pl.BlockSpec((1, d, c_num), lambda e: (e, 0, 0)),
            pl.BlockSpec((1, 1, c_num), lambda e: (e, 0, 0)),
            pl.BlockSpec((1, 1, 1), lambda e: (e, 0, 0)),
        ],
        out_specs=[
            pl.BlockSpec((1, d), lambda e: (0, 0)),
            pl.BlockSpec((1, c_num), lambda e: (0, 0)),
        ],
        out_shape=[
            jax.ShapeDtypeStruct((1, d), f32),
            jax.ShapeDtypeStruct((1, c_num), f32),
        ],
    )(h, kv, ln1_s, ln1_b, wq_l, bq_l, wo, bo, ln2_s, ln2_b,
      w1, b1, w2, b2, lnf_s, lnf_b, head_w, head_b, g3)

    return (latent, logits, g)


# speedup vs baseline: 1.0902x; 1.0902x over previous
import math

import jax
import jax.numpy as jnp
from jax.experimental import pallas as pl
from jax.experimental.pallas import tpu as pltpu

HEADS = 8
NEG = -1e30
SQRT_2_OVER_PI = 0.7978845608028654
BF16 = jnp.bfloat16


def _ln_rows(xb, s, b):
    m = jnp.mean(xb, axis=-1, keepdims=True)
    v = jnp.mean((xb - m) ** 2, axis=-1, keepdims=True)
    return (xb - m) * jax.lax.rsqrt(v + 1e-5) * s + b


def _gelu(u):
    return 0.5 * u * (1.0 + jnp.tanh(SQRT_2_OVER_PI * (u + 0.044715 * u * u * u)))


def _dot(a, b):
    return jnp.dot(a.astype(BF16), b, preferred_element_type=jnp.float32)


def _mha(qsrc, kv_slice, addmask, dh, scale):
    heads = []
    for hd in range(HEADS):
        q = qsrc[:, hd * dh:(hd + 1) * dh] * scale
        k = kv_slice(0, hd)
        v = kv_slice(1, hd)
        s = jax.lax.dot_general(q, k, (((1,), (1,)), ((), ())),
                                preferred_element_type=jnp.float32)
        s = s + addmask
        m = jnp.max(s, axis=-1, keepdims=True)
        p = jnp.exp((s - m).astype(BF16))
        l = jnp.sum(p, axis=-1, keepdims=True, dtype=jnp.float32)
        o = jax.lax.dot_general(p, v, (((1,), (0,)), ((), ())),
                                preferred_element_type=jnp.float32)
        heads.append(o * (1.0 / l))
    return jnp.concatenate(heads, axis=1)


def _mega_body(x_hbm, wr_ref, br_ref, wfc_ref, bfc_ref, temp_ref, cls_ref,
               wp_ref, bp_ref, layer_refs, lnf_refs, head_refs,
               g_ref, lat_ref, log_ref,
               x_s, g_s, h_s, qkv_s, sem,
               *, n, din, d, dh, bt, nt, rt, n_valid, depth, mlp_d):
    e = pl.program_id(0)
    r_full = rt * bt
    f3 = 3 * d
    scale = jnp.asarray(1.0 / math.sqrt(dh), BF16)
    col = jax.lax.broadcasted_iota(jnp.int32, (8, r_full), 1)[0:1]
    addmask = jnp.where(col < n_valid, 0.0, NEG)

    @pl.when(e == 0)
    def _():
        cp = pltpu.make_async_copy(x_hbm, x_s, sem)
        cp.start()
        cp.wait()
        acc = jnp.zeros((1, d), jnp.float32)
        for t in range(nt):
            xb = x_s[t * bt:(t + 1) * bt, :]
            hb = jnp.dot(xb, wr_ref[...], preferred_element_type=jnp.float32)
            hb = jnp.maximum(hb + br_ref[...], 0.0)
            acc = acc + jnp.sum(hb, axis=0, keepdims=True)
        r = acc * (1.0 / n)
        logits = jnp.dot(r, wfc_ref[...], preferred_element_type=jnp.float32)
        logits = (logits + bfc_ref[...]) / temp_ref[...]
        m = jnp.max(logits, axis=-1, keepdims=True)
        ex = jnp.exp(logits - m)
        gv = ex / jnp.sum(ex, axis=-1, keepdims=True)
        g_s[...] = gv
        g_ref[...] = gv

    for t in range(nt):
        xb = x_s[t * bt:(t + 1) * bt, :]
        hb = jnp.dot(xb, wp_ref[0], preferred_element_type=jnp.float32)
        h_s[t * bt:(t + 1) * bt, :] = jnp.maximum(hb + bp_ref[0], 0.0)
    ri = jax.lax.broadcasted_iota(jnp.int32, (bt, d), 0)
    h_s[nt * bt:(nt + 1) * bt, :] = jnp.where(ri == 0, cls_ref[0], 0.0)

    for (ln1s, ln1b, wqkv, bqkv, wo, bo, ln2s, ln2b, w1, b1, w2, b2) \
            in layer_refs[:-1]:
        for t in range(rt):
            y = _ln_rows(h_s[t * bt:(t + 1) * bt, :], ln1s[0], ln1b[0])
            qkv_s[t * bt:(t + 1) * bt, :] = (
                _dot(y, wqkv[0]) + bqkv[0]).astype(BF16)

        def kv_slice(which, hd):
            return qkv_s[:, (1 + which) * d + hd * dh:
                         (1 + which) * d + (hd + 1) * dh]

        for t in range(rt):
            qt = qkv_s[t * bt:(t + 1) * bt, 0:d]
            o = _mha(qt, kv_slice, addmask, dh, scale)
            hb = h_s[t * bt:(t + 1) * bt, :] + _dot(o, wo[0]) + bo[0]
            y = _ln_rows(hb, ln2s[0], ln2b[0])
            u = _gelu(_dot(y, w1[0]) + b1[0])
            h_s[t * bt:(t + 1) * bt, :] = hb + _dot(u, w2[0]) + b2[0]

    (ln1s, ln1b, wqkv, bqkv, wo, bo, ln2s, ln2b, w1, b1, w2, b2) = \
        layer_refs[-1]
    for t in range(rt):
        y = _ln_rows(h_s[t * bt:(t + 1) * bt, :], ln1s[0], ln1b[0])
        qkv_s[t * bt:(t + 1) * bt, d:f3] = (
            _dot(y, wqkv[0][:, d:]) + bqkv[0][:, d:]).astype(BF16)

    def kv_slice2(which, hd):
        return qkv_s[:, (1 + which) * d + hd * dh:
                     (1 + which) * d + (hd + 1) * dh]

    h8 = h_s[nt * bt:nt * bt + 8, :]
    y = _ln_rows(h8, ln1s[0], ln1b[0])
    q = (_dot(y, wqkv[0][:, :d]) + bqkv[0][:, :d]).astype(BF16)
    o = _mha(q, kv_slice2, addmask, dh, scale)
    h8 = h8 + _dot(o, wo[0]) + bo[0]
    y = _ln_rows(h8, ln2s[0], ln2b[0])
    u = _gelu(_dot(y, w1[0]) + b1[0])
    h8 = h8 + _dot(u, w2[0]) + b2[0]

    lnfs, lnfb = lnf_refs
    whd, bhd = head_refs
    ln = _ln_rows(h8, lnfs[0], lnfb[0])
    lat = ln[0:1, :]
    lg = _dot(lat, whd[0]) + bhd[0]
    e_num = g_s.shape[1]
    oh = (jax.lax.broadcasted_iota(jnp.int32, (1, e_num), 1) == e)
    ge = jnp.sum(jnp.where(oh, g_s[...], 0.0), axis=1, keepdims=True)
    contrib_lat = lat * ge
    contrib_log = lg * ge

    @pl.when(e == 0)
    def _():
        lat_ref[...] = contrib_lat
        log_ref[...] = contrib_log

    @pl.when(e > 0)
    def _():
        lat_ref[...] += contrib_lat
        log_ref[...] += contrib_log


def kernel(x, params, temp=1.0):
    b, n, din = x.shape
    experts = params['experts']
    e_num = len(experts)
    d = experts[0]['proj_W'].shape[-1]
    f3 = 3 * d
    dh = d // HEADS
    c_num = experts[0]['head_W'].shape[-1]
    depth = len(experts[0]['layers'])
    mlp_d = experts[0]['layers'][0]['W1'].shape[-1]
    f32 = jnp.float32

    x2 = x.reshape(n, din).astype(BF16)

    bt = 256 if n % 256 == 0 else (128 if n % 128 == 0 else n)
    nt = n // bt
    r_full = nt * bt + bt
    rt = r_full // bt
    n_valid = n + 1

    def st(key, dtype=None):
        arrs = [p[key] for p in experts]
        if dtype is not None:
            arrs = [a.astype(dtype) for a in arrs]
        return jnp.stack(arrs)

    def stl(layer, key, dtype=None):
        arrs = [p['layers'][layer][key] for p in experts]
        if dtype is not None:
            arrs = [a.astype(dtype) for a in arrs]
        return jnp.stack(arrs)

    operands = [
        params['router_proj_W'].astype(BF16),
        params['router_proj_b'].reshape(1, d).astype(f32),
        params['router_fc_W'],
        params['router_fc_b'].reshape(1, e_num).astype(f32),
        jnp.asarray(temp, f32).reshape(1, 1),
        st('cls').reshape(e_num, 1, d),
        st('proj_W', BF16),
        st('proj_b').reshape(e_num, 1, d),
    ]
    in_specs = [
        pl.BlockSpec((din, d), lambda e: (0, 0)),
        pl.BlockSpec((1, d), lambda e: (0, 0)),
        pl.BlockSpec((d, e_num), lambda e: (0, 0)),
        pl.BlockSpec((1, e_num), lambda e: (0, 0)),
        pl.BlockSpec((1, 1), lambda e: (0, 0)),
        pl.BlockSpec((1, 1, d), lambda e: (e, 0, 0)),
        pl.BlockSpec((1, din, d), lambda e: (e, 0, 0)),
        pl.BlockSpec((1, 1, d), lambda e: (e, 0, 0)),
    ]
    n_fixed = len(operands)

    per_layer_counts = []
    for layer in range(depth):
        ops = [
            stl(layer, 'ln1_s').reshape(e_num, 1, d),
            stl(layer, 'ln1_b').reshape(e_num, 1, d),
            stl(layer, 'Wqkv', BF16),
            stl(layer, 'bqkv').reshape(e_num, 1, f3),
            stl(layer, 'Wo', BF16),
            stl(layer, 'bo').reshape(e_num, 1, d),
            stl(layer, 'ln2_s').reshape(e_num, 1, d),
            stl(layer, 'ln2_b').reshape(e_num, 1, d),
            stl(layer, 'W1', BF16),
            stl(layer, 'b1').reshape(e_num, 1, mlp_d),
            stl(layer, 'W2', BF16),
            stl(layer, 'b2').reshape(e_num, 1, d),
        ]
        specs = [
            pl.BlockSpec((1, 1, d), lambda e: (e, 0, 0)),
            pl.BlockSpec((1, 1, d), lambda e: (e, 0, 0)),
            pl.BlockSpec((1, d, f3), lambda e: (e, 0, 0)),
            pl.BlockSpec((1, 1, f3), lambda e: (e, 0, 0)),
            pl.BlockSpec((1, d, d), lambda e: (e, 0, 0)),
            pl.BlockSpec((1, 1, d), lambda e: (e, 0, 0)),
            pl.BlockSpec((1, 1, d), lambda e: (e, 0, 0)),
            pl.BlockSpec((1, 1, d), lambda e: (e, 0, 0)),
            pl.BlockSpec((1, d, mlp_d), lambda e: (e, 0, 0)),
            pl.BlockSpec((1, 1, mlp_d), lambda e: (e, 0, 0)),
            pl.BlockSpec((1, mlp_d, d), lambda e: (e, 0, 0)),
            pl.BlockSpec((1, 1, d), lambda e: (e, 0, 0)),
        ]
        operands += ops
        in_specs += specs
        per_layer_counts.append(len(ops))

    operands += [
        st('lnf_s').reshape(e_num, 1, d),
        st('lnf_b').reshape(e_num, 1, d),
        st('head_W', BF16),
        st('head_b').reshape(e_num, 1, c_num),
    ]
    in_specs += [
        pl.BlockSpec((1, 1, d), lambda e: (e, 0, 0)),
        pl.BlockSpec((1, 1, d), lambda e: (e, 0, 0)),
        pl.BlockSpec((1, d, c_num), lambda e: (e, 0, 0)),
        pl.BlockSpec((1, 1, c_num), lambda e: (e, 0, 0)),
    ]

    def body(*refs):
        x_hbm = refs[0]
        fixed = refs[1:1 + n_fixed]
        idx = 1 + n_fixed
        layer_refs = []
        for layer in range(depth):
            cnt = per_layer_counts[layer]
            layer_refs.append(tuple(refs[idx:idx + cnt]))
            idx += cnt
        lnf_refs = (refs[idx], refs[idx + 1])
        head_refs = (refs[idx + 2], refs[idx + 3])
        idx += 4
        g_ref, lat_ref, log_ref = refs[idx], refs[idx + 1], refs[idx + 2]
        idx += 3
        x_s, g_s, h_s, qkv_s, sem = refs[idx:idx + 5]
        (wr_ref, br_ref, wfc_ref, bfc_ref, temp_ref, cls_ref,
         wp_ref, bp_ref) = fixed
        _mega_body(x_hbm, wr_ref, br_ref, wfc_ref, bfc_ref, temp_ref, cls_ref,
                   wp_ref, bp_ref, layer_refs, lnf_refs, head_refs,
                   g_ref, lat_ref, log_ref, x_s, g_s, h_s, qkv_s, sem,
                   n=float(n), din=din, d=d, dh=dh, bt=bt, nt=nt, rt=rt,
                   n_valid=n_valid, depth=depth, mlp_d=mlp_d)

    g, latent, logits = pl.pallas_call(
        body,
        grid=(e_num,),
        in_specs=[pl.BlockSpec(memory_space=pl.ANY)] + in_specs,
        out_specs=[
            pl.BlockSpec((1, e_num), lambda e: (0, 0)),
            pl.BlockSpec((1, d), lambda e: (0, 0)),
            pl.BlockSpec((1, c_num), lambda e: (0, 0)),
        ],
        out_shape=[
            jax.ShapeDtypeStruct((1, e_num), f32),
            jax.ShapeDtypeStruct((1, d), f32),
            jax.ShapeDtypeStruct((1, c_num), f32),
        ],
        scratch_shapes=[
            pltpu.VMEM((n, din), BF16),
            pltpu.VMEM((1, e_num), f32),
            pltpu.VMEM((r_full, d), f32),
            pltpu.VMEM((r_full, f3), BF16),
            pltpu.SemaphoreType.DMA,
        ],
    )(x2, *operands)

    return (latent, logits, g)


# speedup vs baseline: 1.1335x; 1.0397x over previous
import math

import jax
import jax.numpy as jnp
from jax.experimental import pallas as pl
from jax.experimental.pallas import tpu as pltpu

HEADS = 8
NEG = -1e30
SQRT_2_OVER_PI = 0.7978845608028654
BF16 = jnp.bfloat16


def _ln_rows(xb, s, b):
    m = jnp.mean(xb, axis=-1, keepdims=True)
    v = jnp.mean((xb - m) ** 2, axis=-1, keepdims=True)
    return (xb - m) * jax.lax.rsqrt(v + 1e-5) * s + b


def _gelu(u):
    return 0.5 * u * (1.0 + jnp.tanh(SQRT_2_OVER_PI * (u + 0.044715 * u * u * u)))


def _dot(a, b):
    return jnp.dot(a.astype(BF16), b, preferred_element_type=jnp.float32)


def _mha(qsrc, kv_slice, addmask, dh, scale):
    heads = []
    for hd in range(HEADS):
        q = qsrc[:, hd * dh:(hd + 1) * dh] * scale
        k = kv_slice(0, hd)
        v = kv_slice(1, hd)
        s = jax.lax.dot_general(q, k, (((1,), (1,)), ((), ())),
                                preferred_element_type=jnp.float32)
        s = s + addmask
        m = jnp.max(s, axis=-1, keepdims=True)
        p = jnp.exp((s - m).astype(BF16))
        l = jnp.sum(p, axis=-1, keepdims=True, dtype=jnp.float32)
        o = jax.lax.dot_general(p, v, (((1,), (0,)), ((), ())),
                                preferred_element_type=jnp.float32)
        heads.append(o * (1.0 / l))
    return jnp.concatenate(heads, axis=1)


def _mega_body(x_hbm, wr_ref, br_ref, wfc_ref, bfc_ref, temp_ref, cls_ref,
               wp_ref, bp_ref, layer_refs, lnf_refs, head_refs,
               g_ref, lat_ref, log_ref,
               x_s, g_s, h_s, qkv_s, sem,
               *, n, din, d, dh, bt, nt, rt, n_valid, depth, mlp_d):
    e = pl.program_id(0)
    r_full = rt * bt
    f3 = 3 * d
    c0 = nt * bt
    c8 = c0 + 8
    rk = min(((n_valid + 127) // 128) * 128, r_full)
    scale = jnp.asarray(1.0 / math.sqrt(dh), BF16)
    col = jax.lax.broadcasted_iota(jnp.int32, (8, rk), 1)[0:1]
    addmask = jnp.where(col < n_valid, 0.0, NEG)

    @pl.when(e == 0)
    def _():
        cp = pltpu.make_async_copy(x_hbm, x_s, sem)
        cp.start()
        cp.wait()
        if rk > c8:
            qkv_s[c8:rk, :] = jnp.zeros((rk - c8, f3), BF16)
        acc = jnp.zeros((1, d), jnp.float32)
        for t in range(nt):
            xb = x_s[t * bt:(t + 1) * bt, :]
            hb = jnp.dot(xb, wr_ref[...], preferred_element_type=jnp.float32)
            hb = jnp.maximum(hb + br_ref[...], 0.0)
            acc = acc + jnp.sum(hb, axis=0, keepdims=True)
        r = acc * (1.0 / n)
        logits = jnp.dot(r, wfc_ref[...], preferred_element_type=jnp.float32)
        logits = (logits + bfc_ref[...]) / temp_ref[...]
        m = jnp.max(logits, axis=-1, keepdims=True)
        ex = jnp.exp(logits - m)
        gv = ex / jnp.sum(ex, axis=-1, keepdims=True)
        g_s[...] = gv
        g_ref[...] = gv

    for t in range(nt):
        xb = x_s[t * bt:(t + 1) * bt, :]
        hb = jnp.dot(xb, wp_ref[0], preferred_element_type=jnp.float32)
        h_s[t * bt:(t + 1) * bt, :] = jnp.maximum(hb + bp_ref[0], 0.0)
    ri = jax.lax.broadcasted_iota(jnp.int32, (8, d), 0)
    h_s[c0:c8, :] = jnp.where(ri == 0, cls_ref[0], 0.0)

    def kv_slice(which, hd):
        return qkv_s[0:rk, (1 + which) * d + hd * dh:
                     (1 + which) * d + (hd + 1) * dh]

    for (ln1s, ln1b, wqkv, bqkv, wo, bo, ln2s, ln2b, w1, b1, w2, b2) \
            in layer_refs[:-1]:
        for t in range(nt):
            y = _ln_rows(h_s[t * bt:(t + 1) * bt, :], ln1s[0], ln1b[0])
            qkv_s[t * bt:(t + 1) * bt, :] = (
                _dot(y, wqkv[0]) + bqkv[0]).astype(BF16)
        y = _ln_rows(h_s[c0:c8, :], ln1s[0], ln1b[0])
        qkv_s[c0:c8, :] = (_dot(y, wqkv[0]) + bqkv[0]).astype(BF16)

        def attn_mlp(rows0, rows1):
            qt = qkv_s[rows0:rows1, 0:d]
            o = _mha(qt, kv_slice, addmask, dh, scale)
            hb = h_s[rows0:rows1, :] + _dot(o, wo[0]) + bo[0]
            y2 = _ln_rows(hb, ln2s[0], ln2b[0])
            u = _gelu(_dot(y2, w1[0]) + b1[0])
            h_s[rows0:rows1, :] = hb + _dot(u, w2[0]) + b2[0]

        for t in range(nt):
            attn_mlp(t * bt, (t + 1) * bt)
        attn_mlp(c0, c8)

    (ln1s, ln1b, wqkv, bqkv, wo, bo, ln2s, ln2b, w1, b1, w2, b2) = \
        layer_refs[-1]
    for t in range(nt):
        y = _ln_rows(h_s[t * bt:(t + 1) * bt, :], ln1s[0], ln1b[0])
        qkv_s[t * bt:(t + 1) * bt, d:f3] = (
            _dot(y, wqkv[0][:, d:]) + bqkv[0][:, d:]).astype(BF16)
    y = _ln_rows(h_s[c0:c8, :], ln1s[0], ln1b[0])
    qkv_s[c0:c8, d:f3] = (_dot(y, wqkv[0][:, d:]) + bqkv[0][:, d:]).astype(BF16)

    kv_slice2 = kv_slice

    h8 = h_s[c0:c8, :]
    y = _ln_rows(h8, ln1s[0], ln1b[0])
    q = (_dot(y, wqkv[0][:, :d]) + bqkv[0][:, :d]).astype(BF16)
    o = _mha(q, kv_slice2, addmask, dh, scale)
    h8 = h8 + _dot(o, wo[0]) + bo[0]
    y = _ln_rows(h8, ln2s[0], ln2b[0])
    u = _gelu(_dot(y, w1[0]) + b1[0])
    h8 = h8 + _dot(u, w2[0]) + b2[0]

    lnfs, lnfb = lnf_refs
    whd, bhd = head_refs
    ln = _ln_rows(h8, lnfs[0], lnfb[0])
    lat = ln[0:1, :]
    lg = _dot(lat, whd[0]) + bhd[0]
    e_num = g_s.shape[1]
    oh = (jax.lax.broadcasted_iota(jnp.int32, (1, e_num), 1) == e)
    ge = jnp.sum(jnp.where(oh, g_s[...], 0.0), axis=1, keepdims=True)
    contrib_lat = lat * ge
    contrib_log = lg * ge

    @pl.when(e == 0)
    def _():
        lat_ref[...] = contrib_lat
        log_ref[...] = contrib_log

    @pl.when(e > 0)
    def _():
        lat_ref[...] += contrib_lat
        log_ref[...] += contrib_log


def kernel(x, params, temp=1.0):
    b, n, din = x.shape
    experts = params['experts']
    e_num = len(experts)
    d = experts[0]['proj_W'].shape[-1]
    f3 = 3 * d
    dh = d // HEADS
    c_num = experts[0]['head_W'].shape[-1]
    depth = len(experts[0]['layers'])
    mlp_d = experts[0]['layers'][0]['W1'].shape[-1]
    f32 = jnp.float32

    x2 = x.reshape(n, din).astype(BF16)

    bt = 256 if n % 256 == 0 else (128 if n % 128 == 0 else n)
    nt = n // bt
    r_full = nt * bt + bt
    rt = r_full // bt
    n_valid = n + 1

    def st(key, dtype=None):
        arrs = [p[key] for p in experts]
        if dtype is not None:
            arrs = [a.astype(dtype) for a in arrs]
        return jnp.stack(arrs)

    def stl(layer, key, dtype=None):
        arrs = [p['layers'][layer][key] for p in experts]
        if dtype is not None:
            arrs = [a.astype(dtype) for a in arrs]
        return jnp.stack(arrs)

    operands = [
        params['router_proj_W'].astype(BF16),
        params['router_proj_b'].reshape(1, d).astype(f32),
        params['router_fc_W'],
        params['router_fc_b'].reshape(1, e_num).astype(f32),
        jnp.asarray(temp, f32).reshape(1, 1),
        st('cls').reshape(e_num, 1, d),
        st('proj_W', BF16),
        st('proj_b').reshape(e_num, 1, d),
    ]
    in_specs = [
        pl.BlockSpec((din, d), lambda e: (0, 0)),
        pl.BlockSpec((1, d), lambda e: (0, 0)),
        pl.BlockSpec((d, e_num), lambda e: (0, 0)),
        pl.BlockSpec((1, e_num), lambda e: (0, 0)),
        pl.BlockSpec((1, 1), lambda e: (0, 0)),
        pl.BlockSpec((1, 1, d), lambda e: (e, 0, 0)),
        pl.BlockSpec((1, din, d), lambda e: (e, 0, 0)),
        pl.BlockSpec((1, 1, d), lambda e: (e, 0, 0)),
    ]
    n_fixed = len(operands)

    per_layer_counts = []
    for layer in range(depth):
        ops = [
            stl(layer, 'ln1_s').reshape(e_num, 1, d),
            stl(layer, 'ln1_b').reshape(e_num, 1, d),
            stl(layer, 'Wqkv', BF16),
            stl(layer, 'bqkv').reshape(e_num, 1, f3),
            stl(layer, 'Wo', BF16),
            stl(layer, 'bo').reshape(e_num, 1, d),
            stl(layer, 'ln2_s').reshape(e_num, 1, d),
            stl(layer, 'ln2_b').reshape(e_num, 1, d),
            stl(layer, 'W1', BF16),
            stl(layer, 'b1').reshape(e_num, 1, mlp_d),
            stl(layer, 'W2', BF16),
            stl(layer, 'b2').reshape(e_num, 1, d),
        ]
        specs = [
            pl.BlockSpec((1, 1, d), lambda e: (e, 0, 0)),
            pl.BlockSpec((1, 1, d), lambda e: (e, 0, 0)),
            pl.BlockSpec((1, d, f3), lambda e: (e, 0, 0)),
            pl.BlockSpec((1, 1, f3), lambda e: (e, 0, 0)),
            pl.BlockSpec((1, d, d), lambda e: (e, 0, 0)),
            pl.BlockSpec((1, 1, d), lambda e: (e, 0, 0)),
            pl.BlockSpec((1, 1, d), lambda e: (e, 0, 0)),
            pl.BlockSpec((1, 1, d), lambda e: (e, 0, 0)),
            pl.BlockSpec((1, d, mlp_d), lambda e: (e, 0, 0)),
            pl.BlockSpec((1, 1, mlp_d), lambda e: (e, 0, 0)),
            pl.BlockSpec((1, mlp_d, d), lambda e: (e, 0, 0)),
            pl.BlockSpec((1, 1, d), lambda e: (e, 0, 0)),
        ]
        operands += ops
        in_specs += specs
        per_layer_counts.append(len(ops))

    operands += [
        st('lnf_s').reshape(e_num, 1, d),
        st('lnf_b').reshape(e_num, 1, d),
        st('head_W', BF16),
        st('head_b').reshape(e_num, 1, c_num),
    ]
    in_specs += [
        pl.BlockSpec((1, 1, d), lambda e: (e, 0, 0)),
        pl.BlockSpec((1, 1, d), lambda e: (e, 0, 0)),
        pl.BlockSpec((1, d, c_num), lambda e: (e, 0, 0)),
        pl.BlockSpec((1, 1, c_num), lambda e: (e, 0, 0)),
    ]

    def body(*refs):
        x_hbm = refs[0]
        fixed = refs[1:1 + n_fixed]
        idx = 1 + n_fixed
        layer_refs = []
        for layer in range(depth):
            cnt = per_layer_counts[layer]
            layer_refs.append(tuple(refs[idx:idx + cnt]))
            idx += cnt
        lnf_refs = (refs[idx], refs[idx + 1])
        head_refs = (refs[idx + 2], refs[idx + 3])
        idx += 4
        g_ref, lat_ref, log_ref = refs[idx], refs[idx + 1], refs[idx + 2]
        idx += 3
        x_s, g_s, h_s, qkv_s, sem = refs[idx:idx + 5]
        (wr_ref, br_ref, wfc_ref, bfc_ref, temp_ref, cls_ref,
         wp_ref, bp_ref) = fixed
        _mega_body(x_hbm, wr_ref, br_ref, wfc_ref, bfc_ref, temp_ref, cls_ref,
                   wp_ref, bp_ref, layer_refs, lnf_refs, head_refs,
                   g_ref, lat_ref, log_ref, x_s, g_s, h_s, qkv_s, sem,
                   n=float(n), din=din, d=d, dh=dh, bt=bt, nt=nt, rt=rt,
                   n_valid=n_valid, depth=depth, mlp_d=mlp_d)

    g, latent, logits = pl.pallas_call(
        body,
        grid=(e_num,),
        in_specs=[pl.BlockSpec(memory_space=pl.ANY)] + in_specs,
        out_specs=[
            pl.BlockSpec((1, e_num), lambda e: (0, 0)),
            pl.BlockSpec((1, d), lambda e: (0, 0)),
            pl.BlockSpec((1, c_num), lambda e: (0, 0)),
        ],
        out_shape=[
            jax.ShapeDtypeStruct((1, e_num), f32),
            jax.ShapeDtypeStruct((1, d), f32),
            jax.ShapeDtypeStruct((1, c_num), f32),
        ],
        scratch_shapes=[
            pltpu.VMEM((n, din), BF16),
            pltpu.VMEM((1, e_num), f32),
            pltpu.VMEM((r_full, d), f32),
            pltpu.VMEM((r_full, f3), BF16),
            pltpu.SemaphoreType.DMA,
        ],
    )(x2, *operands)

    return (latent, logits, g)


# speedup vs baseline: 1.1433x; 1.0087x over previous
import math

import jax
import jax.numpy as jnp
from jax.experimental import pallas as pl
from jax.experimental.pallas import tpu as pltpu

HEADS = 8
NEG = -1e30
SQRT_2_OVER_PI = 0.7978845608028654
BF16 = jnp.bfloat16


def _ln_rows(xb, s, b):
    m = jnp.mean(xb, axis=-1, keepdims=True)
    v = jnp.mean((xb - m) ** 2, axis=-1, keepdims=True)
    return (xb - m) * jax.lax.rsqrt(v + 1e-5) * s + b


def _gelu(u):
    return 0.5 * u * (1.0 + jnp.tanh(SQRT_2_OVER_PI * (u + 0.044715 * u * u * u)))


def _dot(a, b):
    return jnp.dot(a.astype(BF16), b, preferred_element_type=jnp.float32)


def _mha(qsrc, kv_slice, addmask, dh, scale):
    heads = []
    for hd in range(HEADS):
        q = qsrc[:, hd * dh:(hd + 1) * dh] * scale
        k = kv_slice(0, hd)
        v = kv_slice(1, hd)
        s = jax.lax.dot_general(q, k, (((1,), (1,)), ((), ())),
                                preferred_element_type=jnp.float32)
        s = s + addmask
        m = jnp.max(s, axis=-1, keepdims=True)
        p = jnp.exp((s - m).astype(BF16))
        l = jnp.sum(p, axis=-1, keepdims=True, dtype=jnp.float32)
        o = jax.lax.dot_general(p, v, (((1,), (0,)), ((), ())),
                                preferred_element_type=jnp.float32)
        heads.append(o * (1.0 / l))
    return jnp.concatenate(heads, axis=1)


def _mega_body(x_hbm, wr_ref, br_ref, wfc_ref, bfc_ref, temp_ref, cls_ref,
               wp_ref, bp_ref, layer_refs, lnf_refs, head_refs,
               g_ref, lat_ref, log_ref,
               x_s, g_s, h_s, qkv_s, sem,
               *, n, din, d, dh, bt, nt, rt, n_valid, depth, mlp_d):
    e = pl.program_id(0)
    r_full = rt * bt
    f3 = 3 * d
    c0 = nt * bt
    c8 = c0 + 8
    rk = min(((n_valid + 127) // 128) * 128, r_full)
    scale = jnp.asarray(1.0 / math.sqrt(dh), BF16)
    col = jax.lax.broadcasted_iota(jnp.int32, (8, rk), 1)[0:1]
    addmask = jnp.where(col < n_valid, 0.0, NEG)

    @pl.when(e == 0)
    def _():
        cp = pltpu.make_async_copy(x_hbm, x_s, sem)
        cp.start()
        cp.wait()
        if rk > c8:
            qkv_s[c8:rk, :] = jnp.zeros((rk - c8, f3), BF16)
        acc = jnp.zeros((1, d), jnp.float32)
        for t in range(nt):
            xb = x_s[t * bt:(t + 1) * bt, :]
            hb = jnp.dot(xb, wr_ref[...], preferred_element_type=jnp.float32)
            hb = jnp.maximum(hb + br_ref[...], 0.0)
            acc = acc + jnp.sum(hb, axis=0, keepdims=True)
        r = acc * (1.0 / n)
        logits = jnp.dot(r, wfc_ref[...], preferred_element_type=jnp.float32)
        logits = (logits + bfc_ref[...]) / temp_ref[...]
        m = jnp.max(logits, axis=-1, keepdims=True)
        ex = jnp.exp(logits - m)
        gv = ex / jnp.sum(ex, axis=-1, keepdims=True)
        g_s[...] = gv
        g_ref[...] = gv

    for t in range(nt):
        xb = x_s[t * bt:(t + 1) * bt, :]
        hb = jnp.dot(xb, wp_ref[0], preferred_element_type=jnp.float32)
        h_s[t * bt:(t + 1) * bt, :] = jnp.maximum(hb + bp_ref[0], 0.0)
    ri = jax.lax.broadcasted_iota(jnp.int32, (8, d), 0)
    h_s[c0:c8, :] = jnp.where(ri == 0, cls_ref[0], 0.0)

    def kv_slice(which, hd):
        return qkv_s[0:rk, (1 + which) * d + hd * dh:
                     (1 + which) * d + (hd + 1) * dh]

    for (ln1s, ln1b, wqkv, bqkv, wo, bo, ln2s, ln2b, w1, b1, w2, b2) \
            in layer_refs[:-1]:
        for t in range(nt):
            y = _ln_rows(h_s[t * bt:(t + 1) * bt, :], ln1s[0], ln1b[0])
            qkv_s[t * bt:(t + 1) * bt, :] = (
                _dot(y, wqkv[0]) + bqkv[0]).astype(BF16)
        y = _ln_rows(h_s[c0:c8, :], ln1s[0], ln1b[0])
        qkv_s[c0:c8, :] = (_dot(y, wqkv[0]) + bqkv[0]).astype(BF16)

        def attn_mlp(rows0, rows1):
            qt = qkv_s[rows0:rows1, 0:d]
            o = _mha(qt, kv_slice, addmask, dh, scale)
            hb = h_s[rows0:rows1, :] + _dot(o, wo[0]) + bo[0]
            y2 = _ln_rows(hb, ln2s[0], ln2b[0])
            u = _gelu(_dot(y2, w1[0]) + b1[0])
            h_s[rows0:rows1, :] = hb + _dot(u, w2[0]) + b2[0]

        ac = 256 if bt % 256 == 0 else bt
        for r0 in range(0, nt * bt, ac):
            attn_mlp(r0, r0 + ac)
        attn_mlp(c0, c8)

    (ln1s, ln1b, wqkv, bqkv, wo, bo, ln2s, ln2b, w1, b1, w2, b2) = \
        layer_refs[-1]
    for t in range(nt):
        y = _ln_rows(h_s[t * bt:(t + 1) * bt, :], ln1s[0], ln1b[0])
        qkv_s[t * bt:(t + 1) * bt, d:f3] = (
            _dot(y, wqkv[0][:, d:]) + bqkv[0][:, d:]).astype(BF16)
    y = _ln_rows(h_s[c0:c8, :], ln1s[0], ln1b[0])
    qkv_s[c0:c8, d:f3] = (_dot(y, wqkv[0][:, d:]) + bqkv[0][:, d:]).astype(BF16)

    kv_slice2 = kv_slice

    h8 = h_s[c0:c8, :]
    y = _ln_rows(h8, ln1s[0], ln1b[0])
    q = (_dot(y, wqkv[0][:, :d]) + bqkv[0][:, :d]).astype(BF16)
    o = _mha(q, kv_slice2, addmask, dh, scale)
    h8 = h8 + _dot(o, wo[0]) + bo[0]
    y = _ln_rows(h8, ln2s[0], ln2b[0])
    u = _gelu(_dot(y, w1[0]) + b1[0])
    h8 = h8 + _dot(u, w2[0]) + b2[0]

    lnfs, lnfb = lnf_refs
    whd, bhd = head_refs
    ln = _ln_rows(h8, lnfs[0], lnfb[0])
    lat = ln[0:1, :]
    lg = _dot(lat, whd[0]) + bhd[0]
    e_num = g_s.shape[1]
    oh = (jax.lax.broadcasted_iota(jnp.int32, (1, e_num), 1) == e)
    ge = jnp.sum(jnp.where(oh, g_s[...], 0.0), axis=1, keepdims=True)
    contrib_lat = lat * ge
    contrib_log = lg * ge

    @pl.when(e == 0)
    def _():
        lat_ref[...] = contrib_lat
        log_ref[...] = contrib_log

    @pl.when(e > 0)
    def _():
        lat_ref[...] += contrib_lat
        log_ref[...] += contrib_log


def kernel(x, params, temp=1.0):
    b, n, din = x.shape
    experts = params['experts']
    e_num = len(experts)
    d = experts[0]['proj_W'].shape[-1]
    f3 = 3 * d
    dh = d // HEADS
    c_num = experts[0]['head_W'].shape[-1]
    depth = len(experts[0]['layers'])
    mlp_d = experts[0]['layers'][0]['W1'].shape[-1]
    f32 = jnp.float32

    x2 = x.reshape(n, din).astype(BF16)

    if n % 512 == 0:
        bt = 512
    elif n % 256 == 0:
        bt = 256
    elif n % 128 == 0:
        bt = 128
    else:
        bt = n
    nt = n // bt
    r_full = nt * bt + bt
    rt = r_full // bt
    n_valid = n + 1
    rs = min(((n_valid + 127) // 128) * 128, r_full)

    def st(key, dtype=None):
        arrs = [p[key] for p in experts]
        if dtype is not None:
            arrs = [a.astype(dtype) for a in arrs]
        return jnp.stack(arrs)

    def stl(layer, key, dtype=None):
        arrs = [p['layers'][layer][key] for p in experts]
        if dtype is not None:
            arrs = [a.astype(dtype) for a in arrs]
        return jnp.stack(arrs)

    operands = [
        params['router_proj_W'].astype(BF16),
        params['router_proj_b'].reshape(1, d).astype(f32),
        params['router_fc_W'],
        params['router_fc_b'].reshape(1, e_num).astype(f32),
        jnp.asarray(temp, f32).reshape(1, 1),
        st('cls').reshape(e_num, 1, d),
        st('proj_W', BF16),
        st('proj_b').reshape(e_num, 1, d),
    ]
    in_specs = [
        pl.BlockSpec((din, d), lambda e: (0, 0)),
        pl.BlockSpec((1, d), lambda e: (0, 0)),
        pl.BlockSpec((d, e_num), lambda e: (0, 0)),
        pl.BlockSpec((1, e_num), lambda e: (0, 0)),
        pl.BlockSpec((1, 1), lambda e: (0, 0)),
        pl.BlockSpec((1, 1, d), lambda e: (e, 0, 0)),
        pl.BlockSpec((1, din, d), lambda e: (e, 0, 0)),
        pl.BlockSpec((1, 1, d), lambda e: (e, 0, 0)),
    ]
    n_fixed = len(operands)

    per_layer_counts = []
    for layer in range(depth):
        ops = [
            stl(layer, 'ln1_s').reshape(e_num, 1, d),
            stl(layer, 'ln1_b').reshape(e_num, 1, d),
            stl(layer, 'Wqkv', BF16),
            stl(layer, 'bqkv').reshape(e_num, 1, f3),
            stl(layer, 'Wo', BF16),
            stl(layer, 'bo').reshape(e_num, 1, d),
            stl(layer, 'ln2_s').reshape(e_num, 1, d),
            stl(layer, 'ln2_b').reshape(e_num, 1, d),
            stl(layer, 'W1', BF16),
            stl(layer, 'b1').reshape(e_num, 1, mlp_d),
            stl(layer, 'W2', BF16),
            stl(layer, 'b2').reshape(e_num, 1, d),
        ]
        specs = [
            pl.BlockSpec((1, 1, d), lambda e: (e, 0, 0)),
            pl.BlockSpec((1, 1, d), lambda e: (e, 0, 0)),
            pl.BlockSpec((1, d, f3), lambda e: (e, 0, 0)),
            pl.BlockSpec((1, 1, f3), lambda e: (e, 0, 0)),
            pl.BlockSpec((1, d, d), lambda e: (e, 0, 0)),
            pl.BlockSpec((1, 1, d), lambda e: (e, 0, 0)),
            pl.BlockSpec((1, 1, d), lambda e: (e, 0, 0)),
            pl.BlockSpec((1, 1, d), lambda e: (e, 0, 0)),
            pl.BlockSpec((1, d, mlp_d), lambda e: (e, 0, 0)),
            pl.BlockSpec((1, 1, mlp_d), lambda e: (e, 0, 0)),
            pl.BlockSpec((1, mlp_d, d), lambda e: (e, 0, 0)),
            pl.BlockSpec((1, 1, d), lambda e: (e, 0, 0)),
        ]
        operands += ops
        in_specs += specs
        per_layer_counts.append(len(ops))

    operands += [
        st('lnf_s').reshape(e_num, 1, d),
        st('lnf_b').reshape(e_num, 1, d),
        st('head_W', BF16),
        st('head_b').reshape(e_num, 1, c_num),
    ]
    in_specs += [
        pl.BlockSpec((1, 1, d), lambda e: (e, 0, 0)),
        pl.BlockSpec((1, 1, d), lambda e: (e, 0, 0)),
        pl.BlockSpec((1, d, c_num), lambda e: (e, 0, 0)),
        pl.BlockSpec((1, 1, c_num), lambda e: (e, 0, 0)),
    ]

    def body(*refs):
        x_hbm = refs[0]
        fixed = refs[1:1 + n_fixed]
        idx = 1 + n_fixed
        layer_refs = []
        for layer in range(depth):
            cnt = per_layer_counts[layer]
            layer_refs.append(tuple(refs[idx:idx + cnt]))
            idx += cnt
        lnf_refs = (refs[idx], refs[idx + 1])
        head_refs = (refs[idx + 2], refs[idx + 3])
        idx += 4
        g_ref, lat_ref, log_ref = refs[idx], refs[idx + 1], refs[idx + 2]
        idx += 3
        x_s, g_s, h_s, qkv_s, sem = refs[idx:idx + 5]
        (wr_ref, br_ref, wfc_ref, bfc_ref, temp_ref, cls_ref,
         wp_ref, bp_ref) = fixed
        _mega_body(x_hbm, wr_ref, br_ref, wfc_ref, bfc_ref, temp_ref, cls_ref,
                   wp_ref, bp_ref, layer_refs, lnf_refs, head_refs,
                   g_ref, lat_ref, log_ref, x_s, g_s, h_s, qkv_s, sem,
                   n=float(n), din=din, d=d, dh=dh, bt=bt, nt=nt, rt=rt,
                   n_valid=n_valid, depth=depth, mlp_d=mlp_d)

    g, latent, logits = pl.pallas_call(
        body,
        grid=(e_num,),
        in_specs=[pl.BlockSpec(memory_space=pl.ANY)] + in_specs,
        out_specs=[
            pl.BlockSpec((1, e_num), lambda e: (0, 0)),
            pl.BlockSpec((1, d), lambda e: (0, 0)),
            pl.BlockSpec((1, c_num), lambda e: (0, 0)),
        ],
        out_shape=[
            jax.ShapeDtypeStruct((1, e_num), f32),
            jax.ShapeDtypeStruct((1, d), f32),
            jax.ShapeDtypeStruct((1, c_num), f32),
        ],
        scratch_shapes=[
            pltpu.VMEM((n, din), BF16),
            pltpu.VMEM((1, e_num), f32),
            pltpu.VMEM((rs, d), f32),
            pltpu.VMEM((rs, f3), BF16),
            pltpu.SemaphoreType.DMA,
        ],
    )(x2, *operands)

    return (latent, logits, g)
